# Initial kernel scaffold; baseline (speedup 1.0000x reference)
import jax
import jax.numpy as jnp
from jax.experimental import pallas as pl


def kernel(edge_index, inputs, W_self, W_neigh, b):
    raise NotImplementedError("write your pallas kernel here")


# speedup vs baseline: 5.2568x; 5.2568x over previous
import jax
import jax.numpy as jnp
from jax import lax
from jax.experimental import pallas as pl
from jax.experimental.pallas import tpu as pltpu
from jax.experimental.pallas import tpu_sc as plsc

N = 10000
D = 128
E = 320000
LANES = 16

NC = 2
NS = 16
NW = NC * NS
EW = E // NW
C = 80
NCH = EW // C
RBIG = 640
RLAST = N - (NS - 1) * RBIG
DBIG = 632
DLAST = N - (NS - 1) * DBIG


def _make_sc_agg(compute_deg):
    mesh = plsc.VectorSubcoreMesh(
        core_axis_name="c", subcore_axis_name="s",
        num_cores=NC, num_subcores=NS)

    out_type = [jax.ShapeDtypeStruct((NC * N, D), jnp.float32)]
    scratch = [
        pltpu.VMEM_SHARED((N, D), jnp.float32),
        pltpu.VMEM((C, D), jnp.float32),
        pltpu.VMEM((C,), jnp.int32),
        pltpu.VMEM((C,), jnp.int32),
        pltpu.SemaphoreType.DMA,
    ]
    if compute_deg:
        out_type.append(jax.ShapeDtypeStruct((NC * N, LANES), jnp.float32))
        scratch += [
            pltpu.VMEM_SHARED((N, LANES), jnp.float32),
            pltpu.VMEM((C, LANES), jnp.float32),
            pltpu.VMEM((DBIG, LANES), jnp.float32),
        ]

    def body(h_hbm, src_hbm, dst_hbm, part_hbm, *rest):
        if compute_deg:
            (deg_hbm, acc_sh, rows_v, srcb, dstb, sem,
             deg_sh, ones16, z16) = rest
        else:
            acc_sh, rows_v, srcb, dstb, sem = rest
        cid = lax.axis_index("c")
        sid = lax.axis_index("s")

        zero = jnp.zeros((LANES,), jnp.float32)

        def zrow(i, carry):
            rows_v[i // (D // LANES), pl.ds((i % (D // LANES)) * LANES, LANES)] = zero
            return carry
        lax.fori_loop(0, C * (D // LANES), zrow, 0)

        rbase = sid * RBIG
        nz = jnp.where(sid < NS - 1, RBIG // C, RLAST // C)
        def zacc(i, carry):
            pltpu.sync_copy(rows_v, acc_sh.at[pl.ds(rbase + i * C, C)])
            return carry
        lax.fori_loop(0, nz, zacc, 0)

        if compute_deg:
            one = jnp.ones((LANES,), jnp.float32)
            def fill16(i, carry):
                ones16[i, :] = one
                return carry
            lax.fori_loop(0, C, fill16, 0)
            def z16f(i, carry):
                z16[i, :] = zero
                return carry
            lax.fori_loop(0, DBIG, z16f, 0)

            @pl.when(sid < NS - 1)
            def _zdeg_big():
                pltpu.sync_copy(z16, deg_sh.at[pl.ds(sid * DBIG, DBIG)])

            @pl.when(sid == NS - 1)
            def _zdeg_last():
                pltpu.sync_copy(z16.at[pl.ds(0, DLAST)],
                                deg_sh.at[pl.ds((NS - 1) * DBIG, DLAST)])

        plsc.subcore_barrier()

        ebase = (cid * NS + sid) * EW
        def step(i, carry):
            eo = ebase + i * C
            pltpu.sync_copy(src_hbm.at[pl.ds(eo, C)], srcb)
            pltpu.sync_copy(dst_hbm.at[pl.ds(eo, C)], dstb)
            pltpu.async_copy(h_hbm.at[srcb], rows_v, sem).wait()
            pltpu.sync_copy(rows_v, acc_sh.at[dstb], add=True)
            if compute_deg:
                pltpu.sync_copy(ones16, deg_sh.at[dstb], add=True)
            return carry
        lax.fori_loop(0, NCH, step, 0)

        plsc.subcore_barrier()

        obase = cid * N

        @pl.when(sid < NS - 1)
        def _copy_big():
            pltpu.sync_copy(acc_sh.at[pl.ds(sid * RBIG, RBIG)],
                            part_hbm.at[pl.ds(obase + sid * RBIG, RBIG)])

        @pl.when(sid == NS - 1)
        def _copy_last():
            pltpu.sync_copy(acc_sh.at[pl.ds((NS - 1) * RBIG, RLAST)],
                            part_hbm.at[pl.ds(obase + (NS - 1) * RBIG, RLAST)])

        if compute_deg:
            @pl.when(sid < NS - 1)
            def _cdeg_big():
                pltpu.sync_copy(deg_sh.at[pl.ds(sid * DBIG, DBIG)],
                                deg_hbm.at[pl.ds(obase + sid * DBIG, DBIG)])

            @pl.when(sid == NS - 1)
            def _cdeg_last():
                pltpu.sync_copy(deg_sh.at[pl.ds((NS - 1) * DBIG, DLAST)],
                                deg_hbm.at[pl.ds(obase + (NS - 1) * DBIG, DLAST)])

    ot = tuple(out_type) if compute_deg else out_type[0]
    return pl.kernel(body, out_type=ot, mesh=mesh,
                     scratch_types=tuple(scratch),
                     compiler_params=pltpu.CompilerParams(
                         use_tc_tiling_on_sc=False))


_sc_agg_deg = _make_sc_agg(True)
_sc_agg = _make_sc_agg(False)


def _tc_layer(h, part, deg, Ws, Wn, bvec):
    def body(h_ref, p_ref, d_ref, ws_ref, wn_ref, b_ref, o_ref):
        p = p_ref[0:N, :] + p_ref[N:2 * N, :]
        dsum = d_ref[0:N, :] + d_ref[N:2 * N, :]
        rdeg = 1.0 / jnp.maximum(dsum[:, 0:1], 1.0)
        agg = p * rdeg
        z = jnp.dot(h_ref[...], ws_ref[...], preferred_element_type=jnp.float32)
        z = z + jnp.dot(agg, wn_ref[...], preferred_element_type=jnp.float32)
        z = z + b_ref[...][None, :]
        nrm = jnp.sqrt(jnp.sum(z * z, axis=1, keepdims=True))
        z = z / jnp.maximum(nrm, 1e-12)
        o_ref[...] = jnp.maximum(z, 0.0)

    return pl.pallas_call(
        body,
        out_shape=jax.ShapeDtypeStruct((N, D), jnp.float32),
    )(h, part, deg, Ws, Wn, bvec)


def kernel(edge_index, inputs, W_self, W_neigh, b):
    src = edge_index[0].astype(jnp.int32)
    dst = edge_index[1].astype(jnp.int32)
    part, deg = _sc_agg_deg(inputs, src, dst)
    h = _tc_layer(inputs, part, deg, W_self[0], W_neigh[0], b[0])
    for l in range(1, 3):
        part = _sc_agg(h, src, dst)
        h = _tc_layer(h, part, deg, W_self[l], W_neigh[l], b[l])
    return h
